# Initial kernel scaffold; baseline (speedup 1.0000x reference)
import jax
import jax.numpy as jnp
from jax import lax
from jax.experimental import pallas as pl
from jax.experimental.pallas import tpu as pltpu
from jax.experimental.pallas import tpu_sc as plsc

N = 10000
E = 320000
D = 128
NC = 2
NS = 16
NW = NC * NS
EPW = E // NW
CH = 80
NCHUNK = EPW // CH
RB = 1000
GRID = N // RB
TPR = N // NS
ZCH = 640


def _sc_edge_body(src_hbm, dst_hbm, ssrc_hbm, sdst_hbm, h_hbm,
                  acc0_hbm, acc1_hbm, den0_hbm, den1_hbm,
                  ssrc_v, sdst_v, src_buf, dst_buf, wbuf, rows,
                  acc_sh, den_sh):
    cid = lax.axis_index("c")
    sid = lax.axis_index("s")
    wid = cid * NS + sid

    def zrow_body(i, _):
        for v in range(D // 16):
            rows[i, pl.ds(v * 16, 16)] = jnp.zeros((16,), jnp.float32)
        return 0
    lax.fori_loop(0, CH, zrow_body, 0)

    def zs_body(i, _):
        ssrc_v[pl.ds(i * 16, 16)] = jnp.zeros((16,), jnp.float32)
        return 0
    lax.fori_loop(0, ZCH // 16, zs_body, 0)

    base_r = sid * TPR
    for k in range(TPR // CH):
        pltpu.sync_copy(rows, acc_sh.at[pl.ds(base_r + k * CH, CH)])
    rem = TPR - (TPR // CH) * CH
    pltpu.sync_copy(rows.at[pl.ds(0, rem)],
                    acc_sh.at[pl.ds(base_r + (TPR // CH) * CH, rem)])
    pltpu.sync_copy(ssrc_v.at[pl.ds(0, ZCH)], den_sh.at[pl.ds(sid * ZCH, ZCH)])

    pltpu.sync_copy(ssrc_hbm, ssrc_v)
    pltpu.sync_copy(sdst_hbm, sdst_v)

    plsc.subcore_barrier()

    def chunk_body(ci, _):
        base = wid * EPW + ci * CH
        pltpu.sync_copy(src_hbm.at[pl.ds(base, CH)], src_buf)
        pltpu.sync_copy(dst_hbm.at[pl.ds(base, CH)], dst_buf.at[0])
        pltpu.sync_copy(h_hbm.at[src_buf], rows)
        for g in range(CH // 16):
            si = src_buf[pl.ds(g * 16, 16)]
            di = dst_buf[0, pl.ds(g * 16, 16)]
            sv = plsc.load_gather(ssrc_v, [si])
            dv = plsc.load_gather(sdst_v, [di])
            e = sv + dv
            e = jnp.where(e >= 0.0, e, 0.2 * e)
            wbuf[0, pl.ds(g * 16, 16)] = jnp.exp(e)

        def row_body(r, _):
            w = wbuf[0, r]
            for v in range(D // 16):
                rows[r, pl.ds(v * 16, 16)] = rows[r, pl.ds(v * 16, 16)] * w
            return 0
        lax.fori_loop(0, CH, row_body, 0, unroll=2)

        pltpu.sync_copy(rows, acc_sh.at[dst_buf.at[0]], add=True)
        pltpu.sync_copy(wbuf.at[0], den_sh.at[dst_buf.at[0]], add=True)
        return 0
    lax.fori_loop(0, NCHUNK, chunk_body, 0)

    plsc.subcore_barrier()

    @pl.when(cid == 0)
    def _():
        pltpu.sync_copy(acc_sh.at[pl.ds(sid * TPR, TPR)],
                        acc0_hbm.at[pl.ds(sid * TPR, TPR)])

    @pl.when(cid == 1)
    def _():
        pltpu.sync_copy(acc_sh.at[pl.ds(sid * TPR, TPR)],
                        acc1_hbm.at[pl.ds(sid * TPR, TPR)])

    @pl.when((cid == 0) & (sid == 0))
    def _():
        pltpu.sync_copy(den_sh.at[pl.ds(0, N)], den0_hbm)

    @pl.when((cid == 1) & (sid == 0))
    def _():
        pltpu.sync_copy(den_sh.at[pl.ds(0, N)], den1_hbm)


_SC_EDGE = pl.kernel(
    _sc_edge_body,
    out_type=[jax.ShapeDtypeStruct((N, D), jnp.float32),
              jax.ShapeDtypeStruct((N, D), jnp.float32),
              jax.ShapeDtypeStruct((N,), jnp.float32),
              jax.ShapeDtypeStruct((N,), jnp.float32)],
    mesh=plsc.VectorSubcoreMesh(core_axis_name="c", subcore_axis_name="s",
                                num_cores=NC, num_subcores=NS),
    scratch_types=[
        pltpu.VMEM((N,), jnp.float32),
        pltpu.VMEM((N,), jnp.float32),
        pltpu.VMEM((CH,), jnp.int32),
        pltpu.VMEM((1, CH), jnp.int32),
        pltpu.VMEM((1, CH), jnp.float32),
        pltpu.VMEM((CH, D), jnp.float32),
        pltpu.VMEM_SHARED((N, D), jnp.float32),
        pltpu.VMEM_SHARED((NS * ZCH,), jnp.float32),
    ],
)


def _tc_front_body(x_ref, w_ref, a_ref, h_ref, s_ref):
    h = jnp.dot(x_ref[...], w_ref[...], preferred_element_type=jnp.float32)
    h_ref[...] = h
    s_ref[...] = jnp.dot(h, a_ref[...], preferred_element_type=jnp.float32)


_TC_FRONT = pl.pallas_call(
    _tc_front_body,
    grid=(GRID,),
    in_specs=[pl.BlockSpec((RB, D), lambda i: (i, 0)),
              pl.BlockSpec((D, D), lambda i: (0, 0)),
              pl.BlockSpec((D, 2), lambda i: (0, 0))],
    out_specs=[pl.BlockSpec((RB, D), lambda i: (i, 0)),
               pl.BlockSpec((RB, 2), lambda i: (i, 0))],
    out_shape=[jax.ShapeDtypeStruct((N, D), jnp.float32),
               jax.ShapeDtypeStruct((N, 2), jnp.float32)],
)


def _tc_mid_body(a0_ref, a1_ref, d0_ref, d1_ref, b_ref, w_ref, a_ref,
                 h_ref, s_ref):
    den = d0_ref[...] + d1_ref[...] + 1e-16
    hin = (a0_ref[...] + a1_ref[...]) / den + b_ref[...]
    hin = jnp.maximum(hin, 0.0)
    h = jnp.dot(hin, w_ref[...], preferred_element_type=jnp.float32)
    h_ref[...] = h
    s_ref[...] = jnp.dot(h, a_ref[...], preferred_element_type=jnp.float32)


_TC_MID = pl.pallas_call(
    _tc_mid_body,
    grid=(GRID,),
    in_specs=[pl.BlockSpec((RB, D), lambda i: (i, 0)),
              pl.BlockSpec((RB, D), lambda i: (i, 0)),
              pl.BlockSpec((RB, 1), lambda i: (i, 0)),
              pl.BlockSpec((RB, 1), lambda i: (i, 0)),
              pl.BlockSpec((1, D), lambda i: (0, 0)),
              pl.BlockSpec((D, D), lambda i: (0, 0)),
              pl.BlockSpec((D, 2), lambda i: (0, 0))],
    out_specs=[pl.BlockSpec((RB, D), lambda i: (i, 0)),
               pl.BlockSpec((RB, 2), lambda i: (i, 0))],
    out_shape=[jax.ShapeDtypeStruct((N, D), jnp.float32),
               jax.ShapeDtypeStruct((N, 2), jnp.float32)],
)


def _tc_final_body(a0_ref, a1_ref, d0_ref, d1_ref, b_ref, out_ref):
    den = d0_ref[...] + d1_ref[...] + 1e-16
    out_ref[...] = (a0_ref[...] + a1_ref[...]) / den + b_ref[...]


_TC_FINAL = pl.pallas_call(
    _tc_final_body,
    grid=(GRID,),
    in_specs=[pl.BlockSpec((RB, D), lambda i: (i, 0)),
              pl.BlockSpec((RB, D), lambda i: (i, 0)),
              pl.BlockSpec((RB, 1), lambda i: (i, 0)),
              pl.BlockSpec((RB, 1), lambda i: (i, 0)),
              pl.BlockSpec((1, D), lambda i: (0, 0))],
    out_specs=pl.BlockSpec((RB, D), lambda i: (i, 0)),
    out_shape=jax.ShapeDtypeStruct((N, D), jnp.float32),
)


def kernel(x, edge_index, W1, a1_src, a1_dst, b1, W2, a2_src, a2_dst, b2):
    src = edge_index[0]
    dst = edge_index[1]
    A1 = jnp.stack([a1_src, a1_dst], axis=1)
    A2 = jnp.stack([a2_src, a2_dst], axis=1)

    h1, s1 = _TC_FRONT(x, W1, A1)
    acc0, acc1, den0, den1 = _SC_EDGE(src, dst, s1[:, 0], s1[:, 1], h1)
    h2, s2 = _TC_MID(acc0, acc1, den0[:, None], den1[:, None],
                     b1[None, :], W2, A2)
    p0, p1, q0, q1 = _SC_EDGE(src, dst, s2[:, 0], s2[:, 1], h2)
    out = _TC_FINAL(p0, p1, q0[:, None], q1[:, None], b2[None, :])
    return out


# speedup vs baseline: 24.0919x; 24.0919x over previous
import jax
import jax.numpy as jnp
from jax import lax
from jax.experimental import pallas as pl
from jax.experimental.pallas import tpu as pltpu
from jax.experimental.pallas import tpu_sc as plsc

N = 10000
E = 320000
D = 128
NC = 2
NS = 16
NW = NC * NS
EPW = E // NW
CH = 80
NCHUNK = EPW // CH
RB = 1000
GRID = N // RB
TPR8 = 624
ZCH = 640


def _sc_edge_body(src_hbm, dst_hbm, ssrc_hbm, sdst_hbm, h_hbm,
                  acc0_hbm, acc1_hbm, den0_hbm, den1_hbm,
                  ssrc_v, sdst_v, src_buf, dst_buf, wbuf, rows,
                  acc_sh, den_sh):
    cid = lax.axis_index("c")
    sid = lax.axis_index("s")
    wid = cid * NS + sid

    def zrow_body(i, _):
        for v in range(D // 16):
            rows[i, pl.ds(v * 16, 16)] = jnp.zeros((16,), jnp.float32)
        return 0
    lax.fori_loop(0, CH, zrow_body, 0)

    def zs_body(i, _):
        ssrc_v[pl.ds(i * 16, 16)] = jnp.zeros((16,), jnp.float32)
        return 0
    lax.fori_loop(0, ZCH // 16, zs_body, 0)

    base_r = sid * TPR8
    for k in range(TPR8 // CH):
        pltpu.sync_copy(rows, acc_sh.at[pl.ds(base_r + k * CH, CH)])
    rem = TPR8 - (TPR8 // CH) * CH
    pltpu.sync_copy(rows.at[pl.ds(0, rem)],
                    acc_sh.at[pl.ds(base_r + (TPR8 // CH) * CH, rem)])

    @pl.when(sid == NS - 1)
    def _():
        pltpu.sync_copy(rows.at[pl.ds(0, N - NS * TPR8)],
                        acc_sh.at[pl.ds(NS * TPR8, N - NS * TPR8)])
    pltpu.sync_copy(ssrc_v.at[pl.ds(0, ZCH)], den_sh.at[pl.ds(sid * ZCH, ZCH)])

    pltpu.sync_copy(ssrc_hbm, ssrc_v)
    pltpu.sync_copy(sdst_hbm, sdst_v)

    plsc.subcore_barrier()

    def chunk_body(ci, _):
        base = wid * EPW + ci * CH
        pltpu.sync_copy(src_hbm.at[pl.ds(base, CH)], src_buf)
        pltpu.sync_copy(dst_hbm.at[pl.ds(base, CH)], dst_buf.at[0])
        pltpu.sync_copy(h_hbm.at[src_buf], rows)
        for g in range(CH // 16):
            si = src_buf[pl.ds(g * 16, 16)]
            di = dst_buf[0, pl.ds(g * 16, 16)]
            sv = plsc.load_gather(ssrc_v, [si])
            dv = plsc.load_gather(sdst_v, [di])
            e = sv + dv
            e = jnp.where(e >= 0.0, e, 0.2 * e)
            ex = jnp.exp(e)
            wbuf[0, pl.ds(g * 16, 16)] = ex
            for l in range(16):
                w = ex[l]
                r = g * 16 + l
                for v in range(D // 16):
                    rows[r, pl.ds(v * 16, 16)] = rows[r, pl.ds(v * 16, 16)] * w

        pltpu.sync_copy(rows, acc_sh.at[dst_buf.at[0]], add=True)
        pltpu.sync_copy(wbuf.at[0], den_sh.at[dst_buf.at[0]], add=True)
        return 0
    lax.fori_loop(0, NCHUNK, chunk_body, 0)

    plsc.subcore_barrier()

    TAIL = N - NS * TPR8

    @pl.when(cid == 0)
    def _():
        pltpu.sync_copy(acc_sh.at[pl.ds(sid * TPR8, TPR8)],
                        acc0_hbm.at[pl.ds(sid * TPR8, TPR8)])

    @pl.when(cid == 1)
    def _():
        pltpu.sync_copy(acc_sh.at[pl.ds(sid * TPR8, TPR8)],
                        acc1_hbm.at[pl.ds(sid * TPR8, TPR8)])

    @pl.when((cid == 0) & (sid == NS - 1))
    def _():
        pltpu.sync_copy(acc_sh.at[pl.ds(NS * TPR8, TAIL)],
                        acc0_hbm.at[pl.ds(NS * TPR8, TAIL)])

    @pl.when((cid == 1) & (sid == NS - 1))
    def _():
        pltpu.sync_copy(acc_sh.at[pl.ds(NS * TPR8, TAIL)],
                        acc1_hbm.at[pl.ds(NS * TPR8, TAIL)])

    @pl.when((cid == 0) & (sid == 0))
    def _():
        pltpu.sync_copy(den_sh.at[pl.ds(0, N)], den0_hbm)

    @pl.when((cid == 1) & (sid == 0))
    def _():
        pltpu.sync_copy(den_sh.at[pl.ds(0, N)], den1_hbm)


_SC_EDGE = pl.kernel(
    _sc_edge_body,
    out_type=[jax.ShapeDtypeStruct((N, D), jnp.float32),
              jax.ShapeDtypeStruct((N, D), jnp.float32),
              jax.ShapeDtypeStruct((N,), jnp.float32),
              jax.ShapeDtypeStruct((N,), jnp.float32)],
    mesh=plsc.VectorSubcoreMesh(core_axis_name="c", subcore_axis_name="s",
                                num_cores=NC, num_subcores=NS),
    compiler_params=pltpu.CompilerParams(use_tc_tiling_on_sc=False,
                                         needs_layout_passes=False),
    scratch_types=[
        pltpu.VMEM((N,), jnp.float32),
        pltpu.VMEM((N,), jnp.float32),
        pltpu.VMEM((CH,), jnp.int32),
        pltpu.VMEM((1, CH), jnp.int32),
        pltpu.VMEM((1, CH), jnp.float32),
        pltpu.VMEM((CH, D), jnp.float32),
        pltpu.VMEM_SHARED((N, D), jnp.float32),
        pltpu.VMEM_SHARED((NS * ZCH,), jnp.float32),
    ],
)


def _tc_front_body(x_ref, w_ref, a_ref, h_ref, s_ref):
    h = jnp.dot(x_ref[...], w_ref[...], preferred_element_type=jnp.float32)
    h_ref[...] = h
    s_ref[...] = jnp.dot(h, a_ref[...], preferred_element_type=jnp.float32)


_TC_FRONT = pl.pallas_call(
    _tc_front_body,
    grid=(GRID,),
    in_specs=[pl.BlockSpec((RB, D), lambda i: (i, 0)),
              pl.BlockSpec((D, D), lambda i: (0, 0)),
              pl.BlockSpec((D, 2), lambda i: (0, 0))],
    out_specs=[pl.BlockSpec((RB, D), lambda i: (i, 0)),
               pl.BlockSpec((RB, 2), lambda i: (i, 0))],
    out_shape=[jax.ShapeDtypeStruct((N, D), jnp.float32),
               jax.ShapeDtypeStruct((N, 2), jnp.float32)],
)


def _tc_mid_body(a0_ref, a1_ref, d0_ref, d1_ref, b_ref, w_ref, a_ref,
                 h_ref, s_ref):
    den = d0_ref[...] + d1_ref[...] + 1e-16
    hin = (a0_ref[...] + a1_ref[...]) / den + b_ref[...]
    hin = jnp.maximum(hin, 0.0)
    h = jnp.dot(hin, w_ref[...], preferred_element_type=jnp.float32)
    h_ref[...] = h
    s_ref[...] = jnp.dot(h, a_ref[...], preferred_element_type=jnp.float32)


_TC_MID = pl.pallas_call(
    _tc_mid_body,
    grid=(GRID,),
    in_specs=[pl.BlockSpec((RB, D), lambda i: (i, 0)),
              pl.BlockSpec((RB, D), lambda i: (i, 0)),
              pl.BlockSpec((RB, 1), lambda i: (i, 0)),
              pl.BlockSpec((RB, 1), lambda i: (i, 0)),
              pl.BlockSpec((1, D), lambda i: (0, 0)),
              pl.BlockSpec((D, D), lambda i: (0, 0)),
              pl.BlockSpec((D, 2), lambda i: (0, 0))],
    out_specs=[pl.BlockSpec((RB, D), lambda i: (i, 0)),
               pl.BlockSpec((RB, 2), lambda i: (i, 0))],
    out_shape=[jax.ShapeDtypeStruct((N, D), jnp.float32),
               jax.ShapeDtypeStruct((N, 2), jnp.float32)],
)


def _tc_final_body(a0_ref, a1_ref, d0_ref, d1_ref, b_ref, out_ref):
    den = d0_ref[...] + d1_ref[...] + 1e-16
    out_ref[...] = (a0_ref[...] + a1_ref[...]) / den + b_ref[...]


_TC_FINAL = pl.pallas_call(
    _tc_final_body,
    grid=(GRID,),
    in_specs=[pl.BlockSpec((RB, D), lambda i: (i, 0)),
              pl.BlockSpec((RB, D), lambda i: (i, 0)),
              pl.BlockSpec((RB, 1), lambda i: (i, 0)),
              pl.BlockSpec((RB, 1), lambda i: (i, 0)),
              pl.BlockSpec((1, D), lambda i: (0, 0))],
    out_specs=pl.BlockSpec((RB, D), lambda i: (i, 0)),
    out_shape=jax.ShapeDtypeStruct((N, D), jnp.float32),
)


def kernel(x, edge_index, W1, a1_src, a1_dst, b1, W2, a2_src, a2_dst, b2):
    src = edge_index[0]
    dst = edge_index[1]
    A1 = jnp.stack([a1_src, a1_dst], axis=1)
    A2 = jnp.stack([a2_src, a2_dst], axis=1)

    h1, s1 = _TC_FRONT(x, W1, A1)
    acc0, acc1, den0, den1 = _SC_EDGE(src, dst, s1[:, 0], s1[:, 1], h1)
    h2, s2 = _TC_MID(acc0, acc1, den0[:, None], den1[:, None],
                     b1[None, :], W2, A2)
    p0, p1, q0, q1 = _SC_EDGE(src, dst, s2[:, 0], s2[:, 1], h2)
    out = _TC_FINAL(p0, p1, q0[:, None], q1[:, None], b2[None, :])
    return out


# speedup vs baseline: 36.2024x; 1.5027x over previous
import jax
import jax.numpy as jnp
from jax import lax
from jax.experimental import pallas as pl
from jax.experimental.pallas import tpu as pltpu
from jax.experimental.pallas import tpu_sc as plsc

N = 10000
E = 320000
D = 128
NC = 2
NS = 16
NW = NC * NS
EPW = E // NW
CH = 80
NCHUNK = EPW // CH
RB = 1000
GRID = N // RB
TPR8 = 624
ZCH = 640


NBUF = 2
BCH = 25
NBLK = NCHUNK // BCH


def _sc_edge_body(src_hbm, dst_hbm, ssrc_hbm, sdst_hbm, h_hbm,
                  acc0_hbm, acc1_hbm, den0_hbm, den1_hbm,
                  ssrc_v, sdst_v, sidx, didx, wbuf, rows0, rows1,
                  acc_sh, den_sh, sem0, sem1):
    cid = lax.axis_index("c")
    sid = lax.axis_index("s")
    wid = cid * NS + sid
    rows = (rows0, rows1)
    sems = (sem0, sem1)

    def zrow_body(i, _):
        for v in range(D // 16):
            rows0[i, pl.ds(v * 16, 16)] = jnp.zeros((16,), jnp.float32)
        return 0
    lax.fori_loop(0, CH, zrow_body, 0)

    def zs_body(i, _):
        ssrc_v[pl.ds(i * 16, 16)] = jnp.zeros((16,), jnp.float32)
        return 0
    lax.fori_loop(0, ZCH // 16, zs_body, 0)

    base_r = sid * TPR8
    for k in range(TPR8 // CH):
        pltpu.sync_copy(rows0, acc_sh.at[pl.ds(base_r + k * CH, CH)])
    rem = TPR8 - (TPR8 // CH) * CH
    pltpu.sync_copy(rows0.at[pl.ds(0, rem)],
                    acc_sh.at[pl.ds(base_r + (TPR8 // CH) * CH, rem)])

    @pl.when(sid == NS - 1)
    def _():
        pltpu.sync_copy(rows0.at[pl.ds(0, N - NS * TPR8)],
                        acc_sh.at[pl.ds(NS * TPR8, N - NS * TPR8)])
    pltpu.sync_copy(ssrc_v.at[pl.ds(0, ZCH)], den_sh.at[pl.ds(sid * ZCH, ZCH)])

    pltpu.sync_copy(ssrc_hbm, ssrc_v)
    pltpu.sync_copy(sdst_hbm, sdst_v)
    for blk in range(2):
        pltpu.sync_copy(src_hbm.at[wid, blk],
                        sidx.at[pl.ds(blk * BCH, BCH)])
        pltpu.sync_copy(dst_hbm.at[wid, blk],
                        didx.at[pl.ds(blk * BCH, BCH)])

    plsc.subcore_barrier()

    def _row_of(ci):
        blk = ci // BCH
        return (blk % 2) * BCH + ci % BCH

    for b in range(NBUF):
        pltpu.make_async_copy(h_hbm.at[sidx.at[b]], rows[b], sems[b]).start()

    def _process(ci, b):
        row = _row_of(ci)
        pltpu.make_async_copy(h_hbm.at[sidx.at[row]], rows[b], sems[b]).wait()

        blk = ci // BCH

        @pl.when((ci % BCH == 0) & (blk >= 1) & (blk < NBLK - 1))
        def _():
            p2 = ((blk + 1) % 2) * BCH
            pltpu.sync_copy(src_hbm.at[wid, blk + 1],
                            sidx.at[pl.ds(p2, BCH)])
            pltpu.sync_copy(dst_hbm.at[wid, blk + 1],
                            didx.at[pl.ds(p2, BCH)])

        for g in range(CH // 16):
            si = sidx[row, pl.ds(g * 16, 16)]
            di = didx[row, pl.ds(g * 16, 16)]
            sv = plsc.load_gather(ssrc_v, [si])
            dv = plsc.load_gather(sdst_v, [di])
            e = sv + dv
            e = jnp.where(e >= 0.0, e, 0.2 * e)
            ex = jnp.exp(e)
            wbuf[b, pl.ds(g * 16, 16)] = ex
            for l in range(16):
                w = ex[l]
                r = g * 16 + l
                for v in range(D // 16):
                    rows[b][r, pl.ds(v * 16, 16)] = \
                        rows[b][r, pl.ds(v * 16, 16)] * w
        pltpu.sync_copy(rows[b], acc_sh.at[didx.at[row]], add=True)
        pltpu.sync_copy(wbuf.at[b], den_sh.at[didx.at[row]], add=True)

    def chunk_body(k, _):
        for b in range(NBUF):
            ci = k * NBUF + b
            _process(ci, b)

            @pl.when(ci + NBUF < NCHUNK)
            def _():
                pltpu.make_async_copy(h_hbm.at[sidx.at[_row_of(ci + NBUF)]],
                                      rows[b], sems[b]).start()
        return 0
    lax.fori_loop(0, NCHUNK // NBUF, chunk_body, 0)
    for t in range(NCHUNK - (NCHUNK // NBUF) * NBUF):
        _process(jnp.int32((NCHUNK // NBUF) * NBUF + t), t)

    plsc.subcore_barrier()

    TAIL = N - NS * TPR8

    @pl.when(cid == 0)
    def _():
        pltpu.sync_copy(acc_sh.at[pl.ds(sid * TPR8, TPR8)],
                        acc0_hbm.at[pl.ds(sid * TPR8, TPR8)])

    @pl.when(cid == 1)
    def _():
        pltpu.sync_copy(acc_sh.at[pl.ds(sid * TPR8, TPR8)],
                        acc1_hbm.at[pl.ds(sid * TPR8, TPR8)])

    @pl.when((cid == 0) & (sid == NS - 1))
    def _():
        pltpu.sync_copy(acc_sh.at[pl.ds(NS * TPR8, TAIL)],
                        acc0_hbm.at[pl.ds(NS * TPR8, TAIL)])

    @pl.when((cid == 1) & (sid == NS - 1))
    def _():
        pltpu.sync_copy(acc_sh.at[pl.ds(NS * TPR8, TAIL)],
                        acc1_hbm.at[pl.ds(NS * TPR8, TAIL)])

    @pl.when((cid == 0) & (sid == 0))
    def _():
        pltpu.sync_copy(den_sh.at[pl.ds(0, N)], den0_hbm)

    @pl.when((cid == 1) & (sid == 0))
    def _():
        pltpu.sync_copy(den_sh.at[pl.ds(0, N)], den1_hbm)


_SC_EDGE = pl.kernel(
    _sc_edge_body,
    out_type=[jax.ShapeDtypeStruct((N, D), jnp.float32),
              jax.ShapeDtypeStruct((N, D), jnp.float32),
              jax.ShapeDtypeStruct((N,), jnp.float32),
              jax.ShapeDtypeStruct((N,), jnp.float32)],
    mesh=plsc.VectorSubcoreMesh(core_axis_name="c", subcore_axis_name="s",
                                num_cores=NC, num_subcores=NS),
    compiler_params=pltpu.CompilerParams(use_tc_tiling_on_sc=False,
                                         needs_layout_passes=False),
    scratch_types=[
        pltpu.VMEM((N,), jnp.float32),
        pltpu.VMEM((N,), jnp.float32),
        pltpu.VMEM((2 * BCH, CH), jnp.int32),
        pltpu.VMEM((2 * BCH, CH), jnp.int32),
        pltpu.VMEM((NBUF, CH), jnp.float32),
        pltpu.VMEM((CH, D), jnp.float32),
        pltpu.VMEM((CH, D), jnp.float32),
        pltpu.VMEM_SHARED((N, D), jnp.float32),
        pltpu.VMEM_SHARED((NS * ZCH,), jnp.float32),
        pltpu.SemaphoreType.DMA,
        pltpu.SemaphoreType.DMA,
    ],
)


def _tc_front_body(x_ref, w_ref, a_ref, h_ref, s_ref):
    h = jnp.dot(x_ref[...], w_ref[...], preferred_element_type=jnp.float32)
    h_ref[...] = h
    s_ref[...] = jnp.dot(h, a_ref[...], preferred_element_type=jnp.float32)


_TC_FRONT = pl.pallas_call(
    _tc_front_body,
    grid=(GRID,),
    in_specs=[pl.BlockSpec((RB, D), lambda i: (i, 0)),
              pl.BlockSpec((D, D), lambda i: (0, 0)),
              pl.BlockSpec((D, 2), lambda i: (0, 0))],
    out_specs=[pl.BlockSpec((RB, D), lambda i: (i, 0)),
               pl.BlockSpec((RB, 2), lambda i: (i, 0))],
    out_shape=[jax.ShapeDtypeStruct((N, D), jnp.float32),
               jax.ShapeDtypeStruct((N, 2), jnp.float32)],
)


def _tc_mid_body(a0_ref, a1_ref, d0_ref, d1_ref, b_ref, w_ref, a_ref,
                 h_ref, s_ref):
    den = d0_ref[...] + d1_ref[...] + 1e-16
    hin = (a0_ref[...] + a1_ref[...]) / den + b_ref[...]
    hin = jnp.maximum(hin, 0.0)
    h = jnp.dot(hin, w_ref[...], preferred_element_type=jnp.float32)
    h_ref[...] = h
    s_ref[...] = jnp.dot(h, a_ref[...], preferred_element_type=jnp.float32)


_TC_MID = pl.pallas_call(
    _tc_mid_body,
    grid=(GRID,),
    in_specs=[pl.BlockSpec((RB, D), lambda i: (i, 0)),
              pl.BlockSpec((RB, D), lambda i: (i, 0)),
              pl.BlockSpec((RB, 1), lambda i: (i, 0)),
              pl.BlockSpec((RB, 1), lambda i: (i, 0)),
              pl.BlockSpec((1, D), lambda i: (0, 0)),
              pl.BlockSpec((D, D), lambda i: (0, 0)),
              pl.BlockSpec((D, 2), lambda i: (0, 0))],
    out_specs=[pl.BlockSpec((RB, D), lambda i: (i, 0)),
               pl.BlockSpec((RB, 2), lambda i: (i, 0))],
    out_shape=[jax.ShapeDtypeStruct((N, D), jnp.float32),
               jax.ShapeDtypeStruct((N, 2), jnp.float32)],
)


def _tc_final_body(a0_ref, a1_ref, d0_ref, d1_ref, b_ref, out_ref):
    den = d0_ref[...] + d1_ref[...] + 1e-16
    out_ref[...] = (a0_ref[...] + a1_ref[...]) / den + b_ref[...]


_TC_FINAL = pl.pallas_call(
    _tc_final_body,
    grid=(GRID,),
    in_specs=[pl.BlockSpec((RB, D), lambda i: (i, 0)),
              pl.BlockSpec((RB, D), lambda i: (i, 0)),
              pl.BlockSpec((RB, 1), lambda i: (i, 0)),
              pl.BlockSpec((RB, 1), lambda i: (i, 0)),
              pl.BlockSpec((1, D), lambda i: (0, 0))],
    out_specs=pl.BlockSpec((RB, D), lambda i: (i, 0)),
    out_shape=jax.ShapeDtypeStruct((N, D), jnp.float32),
)


def kernel(x, edge_index, W1, a1_src, a1_dst, b1, W2, a2_src, a2_dst, b2):
    src = edge_index[0].reshape(NW, NBLK, BCH, CH)
    dst = edge_index[1].reshape(NW, NBLK, BCH, CH)
    A1 = jnp.stack([a1_src, a1_dst], axis=1)
    A2 = jnp.stack([a2_src, a2_dst], axis=1)

    h1, s1 = _TC_FRONT(x, W1, A1)
    acc0, acc1, den0, den1 = _SC_EDGE(src, dst, s1[:, 0], s1[:, 1], h1)
    h2, s2 = _TC_MID(acc0, acc1, den0[:, None], den1[:, None],
                     b1[None, :], W2, A2)
    p0, p1, q0, q1 = _SC_EDGE(src, dst, s2[:, 0], s2[:, 1], h2)
    out = _TC_FINAL(p0, p1, q0[:, None], q1[:, None], b2[None, :])
    return out


# speedup vs baseline: 48.9440x; 1.3520x over previous
import jax
import jax.numpy as jnp
from jax import lax
from jax.experimental import pallas as pl
from jax.experimental.pallas import tpu as pltpu
from jax.experimental.pallas import tpu_sc as plsc

N = 10000
E = 320000
D = 128
NC = 2
NS = 16
NW = NC * NS
EPW = E // NW
CH = 80
NCHUNK = EPW // CH
RB = 1000
GRID = N // RB
TPR8 = 624
ZCH = 640


NBUF = 2
BCH = 25
NBLK = NCHUNK // BCH


def _sc_edge_body(src_hbm, dst_hbm, ssrc_hbm, sdst_hbm, h_hbm,
                  acc0_hbm, acc1_hbm, den0_hbm, den1_hbm,
                  ssrc_v, sdst_v, sidx, didx, wbuf, rows0, rows1,
                  acc_sh, den_sh, sem0, sem1):
    cid = lax.axis_index("c")
    sid = lax.axis_index("s")
    wid = cid * NS + sid
    rows = (rows0, rows1)
    sems = (sem0, sem1)

    def zrow_body(i, _):
        for v in range(D // 16):
            rows0[i, pl.ds(v * 16, 16)] = jnp.zeros((16,), jnp.float32)
        return 0
    lax.fori_loop(0, CH, zrow_body, 0)

    def zs_body(i, _):
        ssrc_v[pl.ds(i * 16, 16)] = jnp.zeros((16,), jnp.float32)
        return 0
    lax.fori_loop(0, ZCH // 16, zs_body, 0)

    base_r = sid * TPR8
    for k in range(TPR8 // CH):
        pltpu.sync_copy(rows0, acc_sh.at[pl.ds(base_r + k * CH, CH)])
    rem = TPR8 - (TPR8 // CH) * CH
    pltpu.sync_copy(rows0.at[pl.ds(0, rem)],
                    acc_sh.at[pl.ds(base_r + (TPR8 // CH) * CH, rem)])

    @pl.when(sid == NS - 1)
    def _():
        pltpu.sync_copy(rows0.at[pl.ds(0, N - NS * TPR8)],
                        acc_sh.at[pl.ds(NS * TPR8, N - NS * TPR8)])
    pltpu.sync_copy(ssrc_v.at[pl.ds(0, ZCH)], den_sh.at[pl.ds(sid * ZCH, ZCH)])

    pltpu.sync_copy(ssrc_hbm, ssrc_v)
    pltpu.sync_copy(sdst_hbm, sdst_v)
    for blk in range(2):
        pltpu.sync_copy(src_hbm.at[wid, blk],
                        sidx.at[pl.ds(blk * BCH, BCH)])
        pltpu.sync_copy(dst_hbm.at[wid, blk],
                        didx.at[pl.ds(blk * BCH, BCH)])

    plsc.subcore_barrier()

    def _row_of(ci):
        blk = ci // BCH
        return (blk % 2) * BCH + ci % BCH

    for b in range(NBUF):
        pltpu.make_async_copy(h_hbm.at[sidx.at[b]], rows[b], sems[b]).start()

    def _process(ci, b):
        row = _row_of(ci)
        pltpu.make_async_copy(h_hbm.at[sidx.at[row]], rows[b], sems[b]).wait()

        blk = ci // BCH

        @pl.when((ci % BCH == 0) & (blk >= 1) & (blk < NBLK - 1))
        def _():
            p2 = ((blk + 1) % 2) * BCH
            pltpu.sync_copy(src_hbm.at[wid, blk + 1],
                            sidx.at[pl.ds(p2, BCH)])
            pltpu.sync_copy(dst_hbm.at[wid, blk + 1],
                            didx.at[pl.ds(p2, BCH)])

        @plsc.parallel_loop(0, CH // 16, 1)
        def _scores(g):
            si = sidx[row, pl.ds(g * 16, 16)]
            di = didx[row, pl.ds(g * 16, 16)]
            sv = plsc.load_gather(ssrc_v, [si])
            dv = plsc.load_gather(sdst_v, [di])
            e = sv + dv
            e = jnp.where(e >= 0.0, e, 0.2 * e)
            wbuf[b, pl.ds(g * 16, 16)] = jnp.exp(e)

        @plsc.parallel_loop(0, CH // 16, 1, unroll=2)
        def _scale(g):
            ex = wbuf[b, pl.ds(g * 16, 16)]
            for l in range(16):
                w = ex[l]
                r = g * 16 + l
                for v in range(D // 16):
                    rows[b][r, pl.ds(v * 16, 16)] = \
                        rows[b][r, pl.ds(v * 16, 16)] * w

        pltpu.sync_copy(rows[b], acc_sh.at[didx.at[row]], add=True)
        pltpu.sync_copy(wbuf.at[b], den_sh.at[didx.at[row]], add=True)

    def chunk_body(k, _):
        for b in range(NBUF):
            ci = k * NBUF + b
            _process(ci, b)

            @pl.when(ci + NBUF < NCHUNK)
            def _():
                pltpu.make_async_copy(h_hbm.at[sidx.at[_row_of(ci + NBUF)]],
                                      rows[b], sems[b]).start()
        return 0
    lax.fori_loop(0, NCHUNK // NBUF, chunk_body, 0)
    for t in range(NCHUNK - (NCHUNK // NBUF) * NBUF):
        _process(jnp.int32((NCHUNK // NBUF) * NBUF + t), t)

    plsc.subcore_barrier()

    TAIL = N - NS * TPR8

    @pl.when(cid == 0)
    def _():
        pltpu.sync_copy(acc_sh.at[pl.ds(sid * TPR8, TPR8)],
                        acc0_hbm.at[pl.ds(sid * TPR8, TPR8)])

    @pl.when(cid == 1)
    def _():
        pltpu.sync_copy(acc_sh.at[pl.ds(sid * TPR8, TPR8)],
                        acc1_hbm.at[pl.ds(sid * TPR8, TPR8)])

    @pl.when((cid == 0) & (sid == NS - 1))
    def _():
        pltpu.sync_copy(acc_sh.at[pl.ds(NS * TPR8, TAIL)],
                        acc0_hbm.at[pl.ds(NS * TPR8, TAIL)])

    @pl.when((cid == 1) & (sid == NS - 1))
    def _():
        pltpu.sync_copy(acc_sh.at[pl.ds(NS * TPR8, TAIL)],
                        acc1_hbm.at[pl.ds(NS * TPR8, TAIL)])

    @pl.when((cid == 0) & (sid == 0))
    def _():
        pltpu.sync_copy(den_sh.at[pl.ds(0, N)], den0_hbm)

    @pl.when((cid == 1) & (sid == 0))
    def _():
        pltpu.sync_copy(den_sh.at[pl.ds(0, N)], den1_hbm)


_SC_EDGE = pl.kernel(
    _sc_edge_body,
    out_type=[jax.ShapeDtypeStruct((N, D), jnp.float32),
              jax.ShapeDtypeStruct((N, D), jnp.float32),
              jax.ShapeDtypeStruct((N,), jnp.float32),
              jax.ShapeDtypeStruct((N,), jnp.float32)],
    mesh=plsc.VectorSubcoreMesh(core_axis_name="c", subcore_axis_name="s",
                                num_cores=NC, num_subcores=NS),
    compiler_params=pltpu.CompilerParams(use_tc_tiling_on_sc=False,
                                         needs_layout_passes=False),
    scratch_types=[
        pltpu.VMEM((N,), jnp.float32),
        pltpu.VMEM((N,), jnp.float32),
        pltpu.VMEM((2 * BCH, CH), jnp.int32),
        pltpu.VMEM((2 * BCH, CH), jnp.int32),
        pltpu.VMEM((NBUF, CH), jnp.float32),
        pltpu.VMEM((CH, D), jnp.float32),
        pltpu.VMEM((CH, D), jnp.float32),
        pltpu.VMEM_SHARED((N, D), jnp.float32),
        pltpu.VMEM_SHARED((NS * ZCH,), jnp.float32),
        pltpu.SemaphoreType.DMA,
        pltpu.SemaphoreType.DMA,
    ],
)


def _tc_front_body(x_ref, w_ref, a_ref, h_ref, s_ref):
    h = jnp.dot(x_ref[...], w_ref[...], preferred_element_type=jnp.float32)
    h_ref[...] = h
    s_ref[...] = jnp.dot(h, a_ref[...], preferred_element_type=jnp.float32)


_TC_FRONT = pl.pallas_call(
    _tc_front_body,
    grid=(GRID,),
    in_specs=[pl.BlockSpec((RB, D), lambda i: (i, 0)),
              pl.BlockSpec((D, D), lambda i: (0, 0)),
              pl.BlockSpec((D, 2), lambda i: (0, 0))],
    out_specs=[pl.BlockSpec((RB, D), lambda i: (i, 0)),
               pl.BlockSpec((RB, 2), lambda i: (i, 0))],
    out_shape=[jax.ShapeDtypeStruct((N, D), jnp.float32),
               jax.ShapeDtypeStruct((N, 2), jnp.float32)],
)


def _tc_mid_body(a0_ref, a1_ref, d0_ref, d1_ref, b_ref, w_ref, a_ref,
                 h_ref, s_ref):
    den = d0_ref[...] + d1_ref[...] + 1e-16
    hin = (a0_ref[...] + a1_ref[...]) / den + b_ref[...]
    hin = jnp.maximum(hin, 0.0)
    h = jnp.dot(hin, w_ref[...], preferred_element_type=jnp.float32)
    h_ref[...] = h
    s_ref[...] = jnp.dot(h, a_ref[...], preferred_element_type=jnp.float32)


_TC_MID = pl.pallas_call(
    _tc_mid_body,
    grid=(GRID,),
    in_specs=[pl.BlockSpec((RB, D), lambda i: (i, 0)),
              pl.BlockSpec((RB, D), lambda i: (i, 0)),
              pl.BlockSpec((RB, 1), lambda i: (i, 0)),
              pl.BlockSpec((RB, 1), lambda i: (i, 0)),
              pl.BlockSpec((1, D), lambda i: (0, 0)),
              pl.BlockSpec((D, D), lambda i: (0, 0)),
              pl.BlockSpec((D, 2), lambda i: (0, 0))],
    out_specs=[pl.BlockSpec((RB, D), lambda i: (i, 0)),
               pl.BlockSpec((RB, 2), lambda i: (i, 0))],
    out_shape=[jax.ShapeDtypeStruct((N, D), jnp.float32),
               jax.ShapeDtypeStruct((N, 2), jnp.float32)],
)


def _tc_final_body(a0_ref, a1_ref, d0_ref, d1_ref, b_ref, out_ref):
    den = d0_ref[...] + d1_ref[...] + 1e-16
    out_ref[...] = (a0_ref[...] + a1_ref[...]) / den + b_ref[...]


_TC_FINAL = pl.pallas_call(
    _tc_final_body,
    grid=(GRID,),
    in_specs=[pl.BlockSpec((RB, D), lambda i: (i, 0)),
              pl.BlockSpec((RB, D), lambda i: (i, 0)),
              pl.BlockSpec((RB, 1), lambda i: (i, 0)),
              pl.BlockSpec((RB, 1), lambda i: (i, 0)),
              pl.BlockSpec((1, D), lambda i: (0, 0))],
    out_specs=pl.BlockSpec((RB, D), lambda i: (i, 0)),
    out_shape=jax.ShapeDtypeStruct((N, D), jnp.float32),
)


def kernel(x, edge_index, W1, a1_src, a1_dst, b1, W2, a2_src, a2_dst, b2):
    src = edge_index[0].reshape(NW, NBLK, BCH, CH)
    dst = edge_index[1].reshape(NW, NBLK, BCH, CH)
    A1 = jnp.stack([a1_src, a1_dst], axis=1)
    A2 = jnp.stack([a2_src, a2_dst], axis=1)

    h1, s1 = _TC_FRONT(x, W1, A1)
    acc0, acc1, den0, den1 = _SC_EDGE(src, dst, s1[:, 0], s1[:, 1], h1)
    h2, s2 = _TC_MID(acc0, acc1, den0[:, None], den1[:, None],
                     b1[None, :], W2, A2)
    p0, p1, q0, q1 = _SC_EDGE(src, dst, s2[:, 0], s2[:, 1], h2)
    out = _TC_FINAL(p0, p1, q0[:, None], q1[:, None], b2[None, :])
    return out
